# Initial kernel scaffold; baseline (speedup 1.0000x reference)
import jax
import jax.numpy as jnp
from jax.experimental import pallas as pl


def kernel(inputs, input_mask, W_left, b_left, W_right, b_right, W_score, b_score):
    raise NotImplementedError("write your pallas kernel here")


# speedup vs baseline: 2.2646x; 2.2646x over previous
import numpy as np
import jax
import jax.numpy as jnp
from jax.experimental import pallas as pl
from jax.experimental.pallas import tpu as pltpu

B, T, D, H = 4, 512, 768, 100
K = 2 * T
HP = 128
S = T * (T + 1) // 2

_starts_np, _ends_np = np.triu_indices(T)
_STARTS = jnp.asarray(_starts_np, dtype=jnp.int32)
_ENDS = jnp.asarray(_ends_np, dtype=jnp.int32)

_NEG_MASKED = np.float32(-1e20)
_NEG_TRI = np.float32(-3e38)


_HI = jax.lax.Precision.HIGHEST


def _span_topk_kernel(x_ref, m_ref, wl_ref, bl_ref, wr_ref, br_ref, ws_ref, bs_ref,
                      idx_ref, logit_ref, mask_ref, prob_ref, sc_ref, key_ref, l_ref):
    f32 = jnp.float32
    x = x_ref[0]
    l_ref[...] = jnp.dot(x, wl_ref[...], preferred_element_type=f32) + bl_ref[...]
    R = jnp.dot(x, wr_ref[...], preferred_element_type=f32) + br_ref[...]
    w = ws_ref[...]
    bs = bs_ref[0, 0]

    TI = 16

    def tile(t, carry):
        Lt = l_ref[pl.ds(t * TI, TI), :]
        tmp = jnp.maximum(Lt[:, None, :] + R[None, :, :], 0.0)
        sc = jnp.dot(tmp.reshape(TI * T, HP), w,
                     preferred_element_type=f32, precision=_HI).reshape(TI, T)
        sc_ref[pl.ds(t * TI, TI), :] = sc + bs
        return carry

    jax.lax.fori_loop(0, T // TI, tile, 0)

    row = jax.lax.broadcasted_iota(jnp.int32, (T, T), 0)
    col = jax.lax.broadcasted_iota(jnp.int32, (T, T), 1)
    upper = col >= row

    m = m_ref[0].astype(f32)
    mprod = jax.lax.dot_general(m, m, (((0,), (0,)), ((), ())),
                                preferred_element_type=f32, precision=_HI)
    scores_m = jnp.where((mprod > 0.0) & upper, sc_ref[...],
                         jnp.where(upper, _NEG_MASKED, _NEG_TRI))
    sc_ref[...] = scores_m

    bits = jax.lax.bitcast_convert_type(scores_m, jnp.int32)
    key_ref[...] = jnp.where(bits >= 0, bits, bits ^ jnp.int32(0x7FFFFFFF))

    Kf = f32(K)
    n_nonneg = jnp.sum(jnp.where(key_ref[...] >= 0, f32(1), f32(0)))
    base = jnp.where(n_nonneg >= Kf, jnp.int32(0), jnp.int32(-2147483648))

    def bit_body(i, t):
        cand = t | jax.lax.shift_left(jnp.int32(1), 30 - i)
        c = jnp.sum(jnp.where(key_ref[...] >= cand, f32(1), f32(0)))
        return jnp.where(c >= Kf, cand, t)

    kth = jax.lax.fori_loop(0, 31, bit_body, base)

    key = key_ref[...]
    sel_gt = key > kth
    sel_eq = key == kth
    n_gt = jnp.sum(jnp.where(sel_gt, f32(1), f32(0)))
    n_eq_need = Kf - n_gt

    triu_f = (row <= col).astype(f32)
    slo_f = (col < row).astype(f32)

    eq_f = jnp.where(sel_eq, f32(1), f32(0))
    eq_cum = jnp.dot(eq_f, triu_f, preferred_element_type=f32, precision=_HI)
    eq_rc = jnp.sum(eq_f, axis=1, keepdims=True)
    eq_off = jnp.dot(slo_f, eq_rc, preferred_element_type=f32, precision=_HI)
    eq_rank = eq_off + eq_cum
    sel_f = jnp.where(sel_gt | (sel_eq & (eq_rank <= n_eq_need)), f32(1), f32(0))

    P = jnp.dot(sel_f, triu_f, preferred_element_type=f32, precision=_HI)
    c_row = jnp.sum(sel_f, axis=1, keepdims=True)
    O = jnp.dot(slo_f, c_row, preferred_element_type=f32, precision=_HI)

    I_f = (row == col).astype(f32)
    O_r = jax.lax.dot_general(O, I_f, (((0,), (0,)), ((), ())),
                              preferred_element_type=f32, precision=_HI)
    c_r = jax.lax.dot_general(c_row, I_f, (((0,), (0,)), ((), ())),
                              preferred_element_type=f32, precision=_HI)

    i_col = jax.lax.broadcasted_iota(jnp.int32, (T, 1), 0).astype(f32)
    KC = 256

    def chunk(ck, carry):
        kk = jax.lax.broadcasted_iota(jnp.int32, (KC, T), 0).astype(f32) + (ck * KC).astype(f32)
        RH = jnp.where((O_r <= kk) & (kk < O_r + c_r), f32(1), f32(0))
        r_k = jnp.dot(RH, i_col, preferred_element_type=f32, precision=_HI)
        O_k = jnp.dot(RH, O, preferred_element_type=f32, precision=_HI)
        k_col = jax.lax.broadcasted_iota(jnp.int32, (KC, 1), 0).astype(f32) + (ck * KC).astype(f32)
        l_k = k_col - O_k

        P_rows = jnp.dot(RH, P, preferred_element_type=f32, precision=_HI)
        sel_rows = jnp.dot(RH, sel_f, preferred_element_type=f32, precision=_HI)
        CH = jnp.where((P_rows == l_k + 1.0) & (sel_rows > 0.5), f32(1), f32(0))
        jj = jax.lax.broadcasted_iota(jnp.int32, (KC, T), 1).astype(f32)
        j_k = jnp.sum(CH * jj, axis=1, keepdims=True)

        sc_rows = jnp.dot(RH, sc_ref[...], preferred_element_type=f32, precision=_HI)
        v_k = jnp.sum(CH * sc_rows, axis=1, keepdims=True)
        mp_rows = jnp.dot(RH, mprod, preferred_element_type=f32, precision=_HI)
        mk_k = jnp.sum(CH * mp_rows, axis=1, keepdims=True)

        s_k = r_k * f32(T) - r_k * (r_k - 1.0) * 0.5 + (j_k - r_k)

        idx_ref[0, pl.ds(ck * KC, KC), :] = s_k.astype(jnp.int32)
        logit = jnp.where(v_k == -jnp.inf, f32(-1.0), v_k)
        logit_ref[0, pl.ds(ck * KC, KC), :] = logit
        mask_ref[0, pl.ds(ck * KC, KC), :] = mk_k
        prob_ref[0, pl.ds(ck * KC, KC), :] = mk_k / (f32(1.0) + jnp.exp(-logit))
        return carry

    jax.lax.fori_loop(0, K // KC, chunk, 0)


def _run_pallas(inputs, mask3, wl, bl, wr, br, ws, bs):
    grid = (B,)
    return pl.pallas_call(
        _span_topk_kernel,
        grid=grid,
        in_specs=[
            pl.BlockSpec((1, T, D), lambda b: (b, 0, 0)),
            pl.BlockSpec((1, 1, T), lambda b: (b, 0, 0)),
            pl.BlockSpec((D, HP), lambda b: (0, 0)),
            pl.BlockSpec((1, HP), lambda b: (0, 0)),
            pl.BlockSpec((D, HP), lambda b: (0, 0)),
            pl.BlockSpec((1, HP), lambda b: (0, 0)),
            pl.BlockSpec((HP, 1), lambda b: (0, 0)),
            pl.BlockSpec((1, 1), lambda b: (0, 0)),
        ],
        out_specs=[
            pl.BlockSpec((1, K, 1), lambda b: (b, 0, 0)),
            pl.BlockSpec((1, K, 1), lambda b: (b, 0, 0)),
            pl.BlockSpec((1, K, 1), lambda b: (b, 0, 0)),
            pl.BlockSpec((1, K, 1), lambda b: (b, 0, 0)),
        ],
        out_shape=[
            jax.ShapeDtypeStruct((B, K, 1), jnp.int32),
            jax.ShapeDtypeStruct((B, K, 1), jnp.float32),
            jax.ShapeDtypeStruct((B, K, 1), jnp.float32),
            jax.ShapeDtypeStruct((B, K, 1), jnp.float32),
        ],
        scratch_shapes=[pltpu.VMEM((T, T), jnp.float32),
                        pltpu.VMEM((T, T), jnp.int32),
                        pltpu.VMEM((T, HP), jnp.float32)],
    )(inputs, mask3, wl, bl, wr, br, ws, bs)


@jax.jit
def kernel(inputs, input_mask, W_left, b_left, W_right, b_right, W_score, b_score):
    f32 = jnp.float32
    bf16 = jnp.bfloat16
    wl = jnp.pad(W_left.astype(bf16), ((0, 0), (0, HP - H)))
    wr = jnp.pad(W_right.astype(bf16), ((0, 0), (0, HP - H)))
    bl = jnp.pad(b_left.astype(f32), (0, HP - H)).reshape(1, HP)
    br = jnp.pad(b_right.astype(f32), (0, HP - H)).reshape(1, HP)
    ws = jnp.pad(W_score.astype(f32), ((0, HP - H), (0, 0)))
    bs = b_score.astype(f32).reshape(1, 1)
    mask3 = input_mask.reshape(B, 1, T)

    idx, logit, mk, prob = _run_pallas(inputs.astype(bf16), mask3, wl, bl, wr, br, ws, bs)

    span_mask = jnp.take(input_mask, _STARTS, axis=1) * jnp.take(input_mask, _ENDS, axis=1)
    top_idx = idx.reshape(B, K)
    return span_mask, top_idx, mk, logit, prob


# speedup vs baseline: 2.3567x; 1.0407x over previous
import numpy as np
import jax
import jax.numpy as jnp
from jax.experimental import pallas as pl
from jax.experimental.pallas import tpu as pltpu

B, T, D, H = 4, 512, 768, 100
K = 2 * T
HP = 128
S = T * (T + 1) // 2

_starts_np, _ends_np = np.triu_indices(T)
_STARTS = jnp.asarray(_starts_np, dtype=jnp.int32)
_ENDS = jnp.asarray(_ends_np, dtype=jnp.int32)

_NEG_MASKED = np.float32(-1e20)
_NEG_TRI = np.float32(-3e38)


_HI = jax.lax.Precision.HIGHEST


def _span_topk_kernel(x_ref, m_ref, wl_ref, bl_ref, wr_ref, br_ref, ws_ref, bs_ref,
                      idx_ref, logit_ref, mask_ref, prob_ref, sc_ref, key_ref, l_ref):
    f32 = jnp.float32
    x = x_ref[0]
    l_ref[...] = jnp.dot(x, wl_ref[...], preferred_element_type=f32) + bl_ref[...]
    R = jnp.dot(x, wr_ref[...], preferred_element_type=f32) + br_ref[...]
    w = ws_ref[...]
    bs = bs_ref[0, 0]

    TI = 16

    def tile(t, carry):
        Lt = l_ref[pl.ds(t * TI, TI), :]
        tmp = jnp.maximum(Lt[:, None, :] + R[None, :, :], 0.0)
        sc = jnp.dot(tmp.reshape(TI * T, HP), w,
                     preferred_element_type=f32, precision=_HI).reshape(TI, T)
        sc_ref[pl.ds(t * TI, TI), :] = sc + bs
        return carry

    jax.lax.fori_loop(0, T // TI, tile, 0)

    row = jax.lax.broadcasted_iota(jnp.int32, (T, T), 0)
    col = jax.lax.broadcasted_iota(jnp.int32, (T, T), 1)
    upper = col >= row

    m = m_ref[0].astype(f32).astype(jnp.bfloat16)
    mprod = jax.lax.dot_general(m, m, (((0,), (0,)), ((), ())),
                                preferred_element_type=f32)
    scores_m = jnp.where((mprod > 0.0) & upper, sc_ref[...],
                         jnp.where(upper, _NEG_MASKED, _NEG_TRI))
    sc_ref[...] = scores_m

    bits = jax.lax.bitcast_convert_type(scores_m, jnp.int32)
    key_ref[...] = jnp.where(bits >= 0, bits, bits ^ jnp.int32(0x7FFFFFFF))

    Kf = f32(K)
    n_nonneg = jnp.sum(jnp.where(key_ref[...] >= 0, f32(1), f32(0)))
    base = jnp.where(n_nonneg >= Kf, jnp.int32(0), jnp.int32(-2147483648))

    def bit_body(i, t):
        cand = t | jax.lax.shift_left(jnp.int32(1), 30 - i)
        c = jnp.sum(jnp.where(key_ref[...] >= cand, f32(1), f32(0)))
        return jnp.where(c >= Kf, cand, t)

    kth = jax.lax.fori_loop(0, 31, bit_body, base)

    key = key_ref[...]
    sel_gt = key > kth
    sel_eq = key == kth
    n_gt = jnp.sum(jnp.where(sel_gt, f32(1), f32(0)))
    n_eq_need = Kf - n_gt

    bf16 = jnp.bfloat16
    triu_b = jnp.where(row <= col, f32(1), f32(0)).astype(bf16)
    slo_f = (col < row).astype(f32)

    eq_b = jnp.where(sel_eq, f32(1), f32(0)).astype(bf16)
    eq_cum = jnp.dot(eq_b, triu_b, preferred_element_type=f32)
    eq_rc = jnp.sum(eq_b.astype(f32), axis=1, keepdims=True)
    eq_off = jnp.dot(slo_f, eq_rc, preferred_element_type=f32, precision=_HI)
    eq_rank = eq_off + eq_cum
    sel_b = jnp.where(sel_gt | (sel_eq & (eq_rank <= n_eq_need)), f32(1), f32(0)).astype(bf16)

    P = jnp.dot(sel_b, triu_b, preferred_element_type=f32)
    c_row = jnp.sum(sel_b.astype(f32), axis=1, keepdims=True)
    O = jnp.dot(slo_f, c_row, preferred_element_type=f32, precision=_HI)
    Pq = jnp.floor(P * f32(1.0 / 256.0))
    Pq_b = Pq.astype(bf16)
    Pr_b = (P - f32(256.0) * Pq).astype(bf16)

    I_f = (row == col).astype(f32)
    O_r = jax.lax.dot_general(O, I_f, (((0,), (0,)), ((), ())),
                              preferred_element_type=f32, precision=_HI)
    c_r = jax.lax.dot_general(c_row, I_f, (((0,), (0,)), ((), ())),
                              preferred_element_type=f32, precision=_HI)

    i_col = jax.lax.broadcasted_iota(jnp.int32, (T, 1), 0).astype(f32)
    KC = 256

    def chunk(ck, carry):
        kk = jax.lax.broadcasted_iota(jnp.int32, (KC, T), 0).astype(f32) + (ck * KC).astype(f32)
        RHm = (O_r <= kk) & (kk < O_r + c_r)
        RH = jnp.where(RHm, f32(1), f32(0))
        RH_b = RH.astype(bf16)
        r_k = jnp.dot(RH, i_col, preferred_element_type=f32, precision=_HI)
        O_k = jnp.dot(RH, O, preferred_element_type=f32, precision=_HI)
        k_col = jax.lax.broadcasted_iota(jnp.int32, (KC, 1), 0).astype(f32) + (ck * KC).astype(f32)
        l_k = k_col - O_k

        P_rows = (f32(256.0) * jnp.dot(RH_b, Pq_b, preferred_element_type=f32)
                  + jnp.dot(RH_b, Pr_b, preferred_element_type=f32))
        col_c = jax.lax.broadcasted_iota(jnp.int32, (KC, T), 1)
        P_prev = jnp.where(col_c == 0, f32(0), pltpu.roll(P_rows, 1, 1))
        CH = jnp.where((P_rows == l_k + 1.0) & (P_prev == l_k), f32(1), f32(0))
        jj = col_c.astype(f32)
        j_k = jnp.sum(CH * jj, axis=1, keepdims=True)

        sc_rows = jnp.dot(RH, sc_ref[...], preferred_element_type=f32, precision=_HI)
        v_k = jnp.sum(CH * sc_rows, axis=1, keepdims=True)
        mk_k = jnp.where(v_k <= f32(-1e19), f32(0), f32(1))

        s_k = r_k * f32(T) - r_k * (r_k - 1.0) * 0.5 + (j_k - r_k)

        idx_ref[0, pl.ds(ck * KC, KC), :] = s_k.astype(jnp.int32)
        logit = jnp.where(v_k == -jnp.inf, f32(-1.0), v_k)
        logit_ref[0, pl.ds(ck * KC, KC), :] = logit
        mask_ref[0, pl.ds(ck * KC, KC), :] = mk_k
        prob_ref[0, pl.ds(ck * KC, KC), :] = mk_k / (f32(1.0) + jnp.exp(-logit))
        return carry

    jax.lax.fori_loop(0, K // KC, chunk, 0)


def _run_pallas(inputs, mask3, wl, bl, wr, br, ws, bs):
    grid = (B,)
    return pl.pallas_call(
        _span_topk_kernel,
        grid=grid,
        in_specs=[
            pl.BlockSpec((1, T, D), lambda b: (b, 0, 0)),
            pl.BlockSpec((1, 1, T), lambda b: (b, 0, 0)),
            pl.BlockSpec((D, HP), lambda b: (0, 0)),
            pl.BlockSpec((1, HP), lambda b: (0, 0)),
            pl.BlockSpec((D, HP), lambda b: (0, 0)),
            pl.BlockSpec((1, HP), lambda b: (0, 0)),
            pl.BlockSpec((HP, 1), lambda b: (0, 0)),
            pl.BlockSpec((1, 1), lambda b: (0, 0)),
        ],
        out_specs=[
            pl.BlockSpec((1, K, 1), lambda b: (b, 0, 0)),
            pl.BlockSpec((1, K, 1), lambda b: (b, 0, 0)),
            pl.BlockSpec((1, K, 1), lambda b: (b, 0, 0)),
            pl.BlockSpec((1, K, 1), lambda b: (b, 0, 0)),
        ],
        out_shape=[
            jax.ShapeDtypeStruct((B, K, 1), jnp.int32),
            jax.ShapeDtypeStruct((B, K, 1), jnp.float32),
            jax.ShapeDtypeStruct((B, K, 1), jnp.float32),
            jax.ShapeDtypeStruct((B, K, 1), jnp.float32),
        ],
        scratch_shapes=[pltpu.VMEM((T, T), jnp.float32),
                        pltpu.VMEM((T, T), jnp.int32),
                        pltpu.VMEM((T, HP), jnp.float32)],
    )(inputs, mask3, wl, bl, wr, br, ws, bs)


@jax.jit
def kernel(inputs, input_mask, W_left, b_left, W_right, b_right, W_score, b_score):
    f32 = jnp.float32
    bf16 = jnp.bfloat16
    wl = jnp.pad(W_left.astype(bf16), ((0, 0), (0, HP - H)))
    wr = jnp.pad(W_right.astype(bf16), ((0, 0), (0, HP - H)))
    bl = jnp.pad(b_left.astype(f32), (0, HP - H)).reshape(1, HP)
    br = jnp.pad(b_right.astype(f32), (0, HP - H)).reshape(1, HP)
    ws = jnp.pad(W_score.astype(f32), ((0, HP - H), (0, 0)))
    bs = b_score.astype(f32).reshape(1, 1)
    mask3 = input_mask.reshape(B, 1, T)

    idx, logit, mk, prob = _run_pallas(inputs.astype(bf16), mask3, wl, bl, wr, br, ws, bs)

    span_mask = jnp.take(input_mask, _STARTS, axis=1) * jnp.take(input_mask, _ENDS, axis=1)
    top_idx = idx.reshape(B, K)
    return span_mask, top_idx, mk, logit, prob
